# baseline (speedup 1.0000x reference)
import functools

import jax
import jax.numpy as jnp
from jax import lax
from jax.experimental import pallas as pl
from jax.experimental.pallas import tpu as pltpu
from jax.experimental.pallas import tpu_sc as plsc

VOCAB = 1000000
EMBED = 64
B = 4096
L = 200

NC = 2
NS = 16
NW = NC * NS

N = B * L
IDX_W = 128
PER_W = N // NW
CH = 512
IR = CH // IDX_W
N_CHUNKS = PER_W // CH
IROWS_W = PER_W // IDX_W


def _body(seq_hbm, lbl_hbm, tok_hbm, seg_hbm, out_hbm,
          idx_v, lbl_v, rows_v, seg_rows_v, sem):
    wid = lax.axis_index("s") * NC + lax.axis_index("c")
    row0 = wid * IROWS_W

    def chunk(ci, _):
        ir0 = row0 + ci * IR
        pltpu.sync_copy(seq_hbm.at[pl.ds(ir0, IR)], idx_v)
        pltpu.sync_copy(lbl_hbm.at[pl.ds(ir0, IR)], lbl_v)
        cps = []
        for j in range(IR):
            cps.append(pltpu.async_copy(
                tok_hbm.at[idx_v.at[j]],
                rows_v.at[pl.ds(j * IDX_W, IDX_W)], sem))
        for j in range(IR):
            cps.append(pltpu.async_copy(
                seg_hbm.at[lbl_v.at[j]],
                seg_rows_v.at[pl.ds(j * IDX_W, IDX_W)], sem))
        for cp in cps:
            cp.wait()

        def add_row(r, _):
            for c in range(EMBED // 16):
                sl = pl.ds(c * 16, 16)
                rows_v[r, sl] = rows_v[r, sl] + seg_rows_v[r, sl]
            return 0

        lax.fori_loop(0, CH, add_row, 0)
        pltpu.sync_copy(rows_v, out_hbm.at[pl.ds(ir0 * IDX_W, CH)])
        return 0

    lax.fori_loop(0, N_CHUNKS, chunk, 0)


@jax.jit
def _run(seq2, lbl2, token_table, segment_table):
    mesh = plsc.VectorSubcoreMesh(core_axis_name="c", subcore_axis_name="s")
    f = pl.kernel(
        _body,
        out_type=jax.ShapeDtypeStruct((N, EMBED), jnp.float32),
        mesh=mesh,
        scratch_types=[
            pltpu.VMEM((IR, IDX_W), jnp.int32),
            pltpu.VMEM((IR, IDX_W), jnp.int32),
            pltpu.VMEM((CH, EMBED), jnp.float32),
            pltpu.VMEM((CH, EMBED), jnp.float32),
            pltpu.SemaphoreType.DMA,
        ],
        compiler_params=pltpu.CompilerParams(use_tc_tiling_on_sc=False),
    )
    return f(seq2, lbl2, token_table, segment_table)


def kernel(sequence, segment_label, token_table, segment_table):
    seq2 = sequence.reshape(N // IDX_W, IDX_W)
    lbl2 = segment_label.reshape(N // IDX_W, IDX_W)
    out = _run(seq2, lbl2, token_table, segment_table)
    return out.reshape(B, L, EMBED)


# speedup vs baseline: 7.8512x; 7.8512x over previous
import functools

import jax
import jax.numpy as jnp
from jax import lax
from jax.experimental import pallas as pl
from jax.experimental.pallas import tpu as pltpu
from jax.experimental.pallas import tpu_sc as plsc

VOCAB = 1000000
EMBED = 64
B = 4096
L = 200

NC = 2
NS = 16
NW = NC * NS

N = B * L
IDX_W = 128
PER_W = N // NW
CH = 512
IR = CH // IDX_W
N_CHUNKS = PER_W // CH
IROWS_W = PER_W // IDX_W

SEG_REP = 2048


def _body(seq_hbm, lbl_hbm, tok_hbm, seg_hbm, out_hbm,
          idx_v, lbl_v, rows_v, seg_rows_v, sem):
    sid = lax.axis_index("s")
    wid = sid * NC + lax.axis_index("c")
    row0 = wid * IROWS_W

    def chunk(ci, _):
        ir0 = row0 + ci * IR
        pltpu.sync_copy(seq_hbm.at[pl.ds(ir0, IR)], idx_v)
        pltpu.sync_copy(lbl_hbm.at[pl.ds(ir0, IR)], lbl_v)
        lanes = lax.iota(jnp.int32, 16)
        for j in range(IR):
            for g in range(IDX_W // 16):
                sl = pl.ds(g * 16, 16)
                koff = wid * CH + j * IDX_W + g * 16
                k = (lanes + koff) & (SEG_REP - 1)
                lbl_v[j, sl] = lbl_v[j, sl] + k * 3
        cps = []
        for j in range(IR):
            cps.append(pltpu.async_copy(
                tok_hbm.at[idx_v.at[j]],
                rows_v.at[pl.ds(j * IDX_W, IDX_W)], sem))
        for j in range(IR):
            cps.append(pltpu.async_copy(
                seg_hbm.at[lbl_v.at[j]],
                seg_rows_v.at[pl.ds(j * IDX_W, IDX_W)], sem))
        for cp in cps:
            cp.wait()

        def add_row(r, _):
            for c in range(EMBED // 16):
                sl = pl.ds(c * 16, 16)
                rows_v[r, sl] = rows_v[r, sl] + seg_rows_v[r, sl]
            return 0

        lax.fori_loop(0, CH, add_row, 0)
        pltpu.sync_copy(rows_v, out_hbm.at[pl.ds(ir0 * IDX_W, CH)])
        return 0

    lax.fori_loop(0, N_CHUNKS, chunk, 0)


@jax.jit
def _run(seq2, lbl2, token_table, segment_table):
    mesh = plsc.VectorSubcoreMesh(core_axis_name="c", subcore_axis_name="s")
    f = pl.kernel(
        _body,
        out_type=jax.ShapeDtypeStruct((N, EMBED), jnp.float32),
        mesh=mesh,
        scratch_types=[
            pltpu.VMEM((IR, IDX_W), jnp.int32),
            pltpu.VMEM((IR, IDX_W), jnp.int32),
            pltpu.VMEM((CH, EMBED), jnp.float32),
            pltpu.VMEM((CH, EMBED), jnp.float32),
            pltpu.SemaphoreType.DMA,
        ],
        compiler_params=pltpu.CompilerParams(use_tc_tiling_on_sc=False),
    )
    return f(seq2, lbl2, token_table, segment_table)


def kernel(sequence, segment_label, token_table, segment_table):
    seq2 = sequence.reshape(N // IDX_W, IDX_W)
    lbl2 = segment_label.reshape(N // IDX_W, IDX_W)
    seg_big = jnp.tile(segment_table, (SEG_REP, 1))
    out = _run(seq2, lbl2, token_table, seg_big)
    return out.reshape(B, L, EMBED)


# speedup vs baseline: 8.4891x; 1.0812x over previous
import jax
import jax.numpy as jnp
from jax import lax
from jax.experimental import pallas as pl
from jax.experimental.pallas import tpu as pltpu
from jax.experimental.pallas import tpu_sc as plsc

VOCAB = 1000000
EMBED = 64
B = 4096
L = 200

NC = 2
NS = 16
NW = NC * NS

N = B * L
IDX_W = 128
PER_W = N // NW
CH = 256
IR = CH // IDX_W
N_CHUNKS = PER_W // CH
IROWS_W = PER_W // IDX_W

SEG_REP = 2048


def _body(seq_hbm, lbl_hbm, tok_hbm, seg_hbm, out_hbm,
          idx0, lbl0, rows0, segr0, idx1, lbl1, rows1, segr1,
          g0, g1, w0, w1):
    wid = lax.axis_index("s") * NC + lax.axis_index("c")
    row0 = wid * IROWS_W
    lanes = lax.iota(jnp.int32, 16)

    bufs = ((idx0, lbl0, rows0, segr0, g0, w0),
            (idx1, lbl1, rows1, segr1, g1, w1))

    def prep(c, bs):
        idx_v, lbl_v, rows_v, segr_v, g, _ = bs
        ir0 = row0 + c * IR
        pltpu.sync_copy(seq_hbm.at[pl.ds(ir0, IR)], idx_v)
        pltpu.sync_copy(lbl_hbm.at[pl.ds(ir0, IR)], lbl_v)
        for j in range(IR):
            for q in range(IDX_W // 16):
                sl = pl.ds(q * 16, 16)
                koff = wid * CH + j * IDX_W + q * 16
                lbl_v[j, sl] = lbl_v[j, sl] + ((lanes + koff) & (SEG_REP - 1)) * 3
        for j in range(IR):
            pltpu.async_copy(tok_hbm.at[idx_v.at[j]],
                             rows_v.at[pl.ds(j * IDX_W, IDX_W)], g)
            pltpu.async_copy(seg_hbm.at[lbl_v.at[j]],
                             segr_v.at[pl.ds(j * IDX_W, IDX_W)], g)

    def finish(c, bs):
        idx_v, lbl_v, rows_v, segr_v, g, w = bs
        pltpu.make_async_copy(out_hbm.at[pl.ds(0, CH)], rows_v, g).wait()
        pltpu.make_async_copy(out_hbm.at[pl.ds(0, CH)], rows_v, g).wait()

        def add_row(r, _):
            for q in range(EMBED // 16):
                sl = pl.ds(q * 16, 16)
                rows_v[r, sl] = rows_v[r, sl] + segr_v[r, sl]
            return 0

        lax.fori_loop(0, CH, add_row, 0)
        pltpu.async_copy(rows_v, out_hbm.at[pl.ds((row0 + c * IR) * IDX_W, CH)], w)

    def drain_w(bs):
        _, _, rows_v, _, _, w = bs
        pltpu.make_async_copy(rows_v, out_hbm.at[pl.ds(0, CH)], w).wait()

    prep(0, bufs[0])

    def iter_t(t, _):
        c0 = 2 * t
        c1 = c0 + 1

        @pl.when(t != 0)
        def _():
            drain_w(bufs[1])

        prep(c1, bufs[1])
        finish(c0, bufs[0])

        drain_w(bufs[0])

        @pl.when(c1 + 1 < N_CHUNKS)
        def _():
            prep(c1 + 1, bufs[0])

        finish(c1, bufs[1])
        return 0

    lax.fori_loop(0, N_CHUNKS // 2, iter_t, 0)
    drain_w(bufs[1])


@jax.jit
def _run(seq2, lbl2, token_table, seg_big):
    mesh = plsc.VectorSubcoreMesh(core_axis_name="c", subcore_axis_name="s")
    f = pl.kernel(
        _body,
        out_type=jax.ShapeDtypeStruct((N, EMBED), jnp.float32),
        mesh=mesh,
        scratch_types=[
            pltpu.VMEM((IR, IDX_W), jnp.int32),
            pltpu.VMEM((IR, IDX_W), jnp.int32),
            pltpu.VMEM((CH, EMBED), jnp.float32),
            pltpu.VMEM((CH, EMBED), jnp.float32),
            pltpu.VMEM((IR, IDX_W), jnp.int32),
            pltpu.VMEM((IR, IDX_W), jnp.int32),
            pltpu.VMEM((CH, EMBED), jnp.float32),
            pltpu.VMEM((CH, EMBED), jnp.float32),
            pltpu.SemaphoreType.DMA,
            pltpu.SemaphoreType.DMA,
            pltpu.SemaphoreType.DMA,
            pltpu.SemaphoreType.DMA,
        ],
        compiler_params=pltpu.CompilerParams(use_tc_tiling_on_sc=False),
    )
    return f(seq2, lbl2, token_table, seg_big)


def kernel(sequence, segment_label, token_table, segment_table):
    seq2 = sequence.reshape(N // IDX_W, IDX_W)
    lbl2 = segment_label.reshape(N // IDX_W, IDX_W)
    seg_big = jnp.tile(segment_table, (SEG_REP, 1))
    out = _run(seq2, lbl2, token_table, seg_big)
    return out.reshape(B, L, EMBED)
